# baseline (speedup 1.0000x reference)
import jax
import jax.numpy as jnp
from jax.experimental import pallas as pl
from jax.experimental.pallas import tpu as pltpu

_TB = 512


def _toeplitz(vfull, J, S):
    L = vfull.shape[0]
    stack = jnp.pad(vfull, ((0, S + 1 - L),) + ((0, 0),) * (vfull.ndim - 1))
    reps = -(-(J * S) // (S + 1)) + 1
    flat = jnp.tile(stack, (reps,) + (1,) * (vfull.ndim - 1))
    return flat[:J * S].reshape((J, S) + vfull.shape[1:])


def _fused_kernel(x_ref, w1t_ref, b1t_ref, w2t_ref, b2t_ref, wf1t_ref,
                  bf1t_ref, wf2t_ref, bf2t_ref, o_ref, p1_ref, feat_ref):
    f32 = jnp.float32
    bf16 = jnp.bfloat16

    for ho in range(15):
        xs = jnp.concatenate(
            [x_ref[pl.ds(1040 * ci + 64 * ho, 144), :] for ci in range(3)],
            axis=0)
        out = jnp.dot(w1t_ref[...], xs, preferred_element_type=f32)
        out = jnp.maximum(out + b1t_ref[...], 0.0)
        m = jnp.maximum(out[:640, :], out[640:, :])
        m = m.reshape(16, 2, 20, _TB)
        m = jnp.maximum(m[:, 0], m[:, 1])
        m = m[:15].reshape(300, _TB)
        p1_ref[pl.ds(300 * ho, 300), :] = m

    for h2 in range(5):
        ps = p1_ref[pl.ds(600 * h2, 1880), :].astype(bf16)
        out = jnp.dot(w2t_ref[...], ps, preferred_element_type=f32)
        out = jnp.maximum(out + b2t_ref[...], 0.0)
        m = jnp.maximum(out[:750, :], out[750:, :])
        m = m[:500].reshape(5, 2, 50, _TB)
        m = jnp.maximum(m[:, 0], m[:, 1])
        feat_ref[pl.ds(250 * h2, 250), :] = m.reshape(250, _TB)

    f = feat_ref[...].astype(bf16)
    h = jnp.dot(wf1t_ref[...], f, preferred_element_type=f32) + bf1t_ref[...]
    h = jnp.maximum(h, 0.0).astype(bf16)
    o_ref[...] = jnp.dot(wf2t_ref[...], h,
                         preferred_element_type=f32) + bf2t_ref[...]


def kernel(x, w1, b1, w2, b2, wf1, bf1, wf2, bf2):
    B = x.shape[0]
    C = wf2.shape[1]
    Bp = -(-B // _TB) * _TB

    x2 = jnp.pad(x.astype(jnp.bfloat16).reshape(B, 3, 1024),
                 ((0, 0), (0, 0), (0, 16))).reshape(B, 3120).T
    if Bp != B:
        x2 = jnp.pad(x2, ((0, 0), (0, Bp - B)))

    vf1 = jnp.pad(w1.astype(jnp.bfloat16).reshape(3, 3, 3, 20), ((0, 0), (0, 29), (0, 0), (0, 0)))
    vf1 = vf1.reshape(96, 3, 20)[:67]
    t1 = _toeplitz(vf1, 64, 130)
    w1t = jnp.pad(t1.transpose(0, 3, 2, 1), ((0, 0), (0, 0), (0, 0), (0, 14)))
    w1t = w1t.reshape(1280, 432)
    b1t = jnp.broadcast_to(jnp.tile(b1[0].astype(jnp.bfloat16), 64)[:, None], (1280, _TB))

    vf2 = jnp.pad(w2.astype(jnp.bfloat16).reshape(5, 5, 20, 50), ((0, 0), (0, 10), (0, 0), (0, 0)))
    vf2 = vf2.reshape(75, 20, 50)[:65]
    t2 = _toeplitz(vf2, 30, 94)
    w2t = t2.transpose(0, 3, 1, 2).reshape(1500, 1880)
    b2t = jnp.broadcast_to(jnp.tile(b2[0].astype(jnp.bfloat16), 30)[:, None], (1500, _TB))

    wf1t = jnp.pad(wf1.astype(jnp.bfloat16).T, ((0, 512 - wf1.shape[1]), (0, 0)))
    bf1t = jnp.pad(jnp.broadcast_to(bf1[0].astype(jnp.bfloat16)[:, None], (wf1.shape[1], _TB)),
                   ((0, 512 - wf1.shape[1]), (0, 0)))
    wf2t = jnp.pad(wf2.astype(jnp.bfloat16).T, ((0, 16 - C), (0, 512 - wf2.shape[0])))
    bf2t = jnp.pad(jnp.broadcast_to(bf2[0].astype(jnp.bfloat16)[:, None], (C, _TB)), ((0, 16 - C), (0, 0)))

    grid = Bp // _TB
    flops = 2 * grid * _TB * (15 * 1280 * 392 + 5 * 1500 * 1880
                              + 512 * 1250 + 16 * 512) // _TB * _TB
    bytes_accessed = 4 * (Bp * 3120 + Bp * 16 + w1t.size + w2t.size
                          + wf1t.size + b1t.size + b2t.size)

    out = pl.pallas_call(
        _fused_kernel,
        out_shape=jax.ShapeDtypeStruct((16, Bp), jnp.float32),
        grid=(grid,),
        in_specs=[
            pl.BlockSpec((3120, _TB), lambda i: (0, i)),
            pl.BlockSpec((1280, 432), lambda i: (0, 0)),
            pl.BlockSpec((1280, _TB), lambda i: (0, 0)),
            pl.BlockSpec((1500, 1880), lambda i: (0, 0)),
            pl.BlockSpec((1500, _TB), lambda i: (0, 0)),
            pl.BlockSpec((512, 1250), lambda i: (0, 0)),
            pl.BlockSpec((512, _TB), lambda i: (0, 0)),
            pl.BlockSpec((16, 512), lambda i: (0, 0)),
            pl.BlockSpec((16, _TB), lambda i: (0, 0)),
        ],
        out_specs=pl.BlockSpec((16, _TB), lambda i: (0, i)),
        scratch_shapes=[
            pltpu.VMEM((4500, _TB), jnp.float32),
            pltpu.VMEM((1250, _TB), jnp.float32),
        ],
        compiler_params=pltpu.CompilerParams(
            dimension_semantics=("parallel",),
            vmem_limit_bytes=100 * 1024 * 1024),
        cost_estimate=pl.CostEstimate(flops=flops, transcendentals=0,
                                      bytes_accessed=bytes_accessed),
    )(x2, w1t, b1t, w2t, b2t, wf1t, bf1t, wf2t, bf2t)
    return out[:C, :B].T


# speedup vs baseline: 1.0936x; 1.0936x over previous
import jax
import jax.numpy as jnp
from jax.experimental import pallas as pl
from jax.experimental.pallas import tpu as pltpu

_TB = 256


def _toeplitz(vfull, J, S):
    L = vfull.shape[0]
    stack = jnp.pad(vfull, ((0, S + 1 - L),) + ((0, 0),) * (vfull.ndim - 1))
    reps = -(-(J * S) // (S + 1)) + 1
    flat = jnp.tile(stack, (reps,) + (1,) * (vfull.ndim - 1))
    return flat[:J * S].reshape((J, S) + vfull.shape[1:])


def _fused_kernel(x_ref, w1t_ref, w2t_ref, b2t_ref, wf1t_ref, wf2t_ref,
                  bf2t_ref, o_ref, p1_ref, feat_ref):
    f32 = jnp.float32
    bf16 = jnp.bfloat16
    zpad = jnp.zeros((16, _TB), bf16)
    ones8 = jnp.ones((8, _TB), bf16)

    for ho in range(15):
        if ho < 14:
            parts = [x_ref[pl.ds(1024 * ci + 64 * ho, 144), :] for ci in range(3)]
        else:
            parts = []
            for ci in range(3):
                parts += [x_ref[pl.ds(1024 * ci + 896, 128), :], zpad]
        xs = jnp.concatenate(parts + [ones8], axis=0)
        out = jnp.dot(w1t_ref[...], xs, preferred_element_type=f32)
        out = jnp.maximum(out, 0.0)
        m = jnp.maximum(out[:640, :], out[640:, :])
        m = m.reshape(16, 2, 20, _TB)
        m = jnp.maximum(m[:, 0], m[:, 1])
        m = m[:15].reshape(300, _TB)
        p1_ref[pl.ds(300 * ho, 300), :] = m

    for h2 in range(5):
        ps = p1_ref[pl.ds(600 * h2, 1880), :].astype(bf16)
        out = jnp.dot(w2t_ref[...], ps, preferred_element_type=f32)
        out = jnp.maximum(out + b2t_ref[...], 0.0)
        m = jnp.maximum(out[:750, :], out[750:, :])
        m = m[:500].reshape(5, 2, 50, _TB)
        m = jnp.maximum(m[:, 0], m[:, 1])
        feat_ref[pl.ds(250 * h2, 250), :] = m.reshape(250, _TB)

    feat_ref[pl.ds(1250, 14), :] = (
        jax.lax.broadcasted_iota(jnp.int32, (14, _TB), 0) == 0).astype(f32)

    f = feat_ref[...].astype(bf16)
    h = jnp.maximum(jnp.dot(wf1t_ref[...], f, preferred_element_type=f32), 0.0)
    o_ref[...] = jnp.dot(wf2t_ref[...], h.astype(bf16),
                         preferred_element_type=f32) + bf2t_ref[...]


def kernel(x, w1, b1, w2, b2, wf1, bf1, wf2, bf2):
    B = x.shape[0]
    C = wf2.shape[1]
    Bp = -(-B // _TB) * _TB
    bf16 = jnp.bfloat16

    x2 = x.astype(bf16).reshape(B, 3072).T
    if Bp != B:
        x2 = jnp.pad(x2, ((0, 0), (0, Bp - B)))

    vf1 = jnp.pad(w1.astype(bf16).reshape(3, 3, 3, 20),
                  ((0, 0), (0, 29), (0, 0), (0, 0)))
    vf1 = vf1.reshape(96, 3, 20)[:67]
    t1 = _toeplitz(vf1, 64, 130)
    w1t = jnp.pad(t1.transpose(0, 3, 2, 1), ((0, 0), (0, 0), (0, 0), (0, 14)))
    b1col = jnp.pad(jnp.tile(b1.astype(bf16)[0], 64)[:, None], ((0, 0), (0, 7)))
    w1t = jnp.concatenate([w1t.reshape(1280, 432), b1col], axis=1)

    vf2 = jnp.pad(w2.astype(bf16).reshape(5, 5, 20, 50),
                  ((0, 0), (0, 10), (0, 0), (0, 0)))
    vf2 = vf2.reshape(75, 20, 50)[:65]
    t2 = _toeplitz(vf2, 30, 94)
    w2t = t2.transpose(0, 3, 1, 2).reshape(1500, 1880)
    b2t = jnp.broadcast_to(jnp.tile(b2[0].astype(bf16), 30)[:, None], (1500, _TB))

    wf1t = jnp.pad(wf1.astype(bf16).T, ((0, 512 - wf1.shape[1]), (0, 0)))
    bf1col = jnp.pad(bf1.astype(bf16).T, ((0, 512 - wf1.shape[1]), (0, 13)))
    wf1t = jnp.concatenate([wf1t, bf1col], axis=1)
    wf2t = jnp.pad(wf2.astype(bf16).T, ((0, 16 - C), (0, 512 - wf2.shape[0])))
    bf2t = jnp.pad(jnp.broadcast_to(bf2[0].astype(bf16)[:, None], (C, _TB)),
                   ((0, 16 - C), (0, 0)))

    grid = Bp // _TB
    flops = 2 * Bp * (1280 * 440 * 15 + 1500 * 1880 * 5 + 512 * 1264 + 16 * 512) // 64
    bytes_accessed = 2 * (Bp * 3072 + w1t.size + w2t.size + wf1t.size
                          + b2t.size) + 4 * Bp * 16

    out = pl.pallas_call(
        _fused_kernel,
        out_shape=jax.ShapeDtypeStruct((16, Bp), jnp.float32),
        grid=(grid,),
        in_specs=[
            pl.BlockSpec((3072, _TB), lambda i: (0, i)),
            pl.BlockSpec((1280, 440), lambda i: (0, 0)),
            pl.BlockSpec((1500, 1880), lambda i: (0, 0)),
            pl.BlockSpec((1500, _TB), lambda i: (0, 0)),
            pl.BlockSpec((512, 1264), lambda i: (0, 0)),
            pl.BlockSpec((16, 512), lambda i: (0, 0)),
            pl.BlockSpec((16, _TB), lambda i: (0, 0)),
        ],
        out_specs=pl.BlockSpec((16, _TB), lambda i: (0, i)),
        scratch_shapes=[
            pltpu.VMEM((4500, _TB), jnp.float32),
            pltpu.VMEM((1264, _TB), jnp.float32),
        ],
        compiler_params=pltpu.CompilerParams(
            dimension_semantics=("parallel",),
            vmem_limit_bytes=100 * 1024 * 1024),
        cost_estimate=pl.CostEstimate(flops=flops, transcendentals=0,
                                      bytes_accessed=bytes_accessed),
    )(x2, w1t, w2t, b2t, wf1t, wf2t, bf2t)
    return out[:C, :B].T
